# baseline (speedup 1.0000x reference)
import functools

import jax
import jax.numpy as jnp
from jax import lax
from jax.experimental import pallas as pl
from jax.experimental.pallas import tpu as pltpu
from jax.experimental.pallas import tpu_sc as plsc

N = 10000
DH = 128
E = 320000
NC = 2
NS = 16
NW = NC * NS
K = 128
EPW = 10240
PAD_E = NW * EPW - E
NCHUNK = EPW // K
SSTG = 16
NSTG = NCHUNK // SSTG
NPAD = 10112
NPADD = 10240
DEG_SLICE = NPADD // NS
ROWS_PER_TILE = NPAD // NS

R = 2000
GRID = N // R

_sc_mesh = plsc.VectorSubcoreMesh(core_axis_name="c", subcore_axis_name="s")


@functools.partial(
    pl.kernel,
    out_type=jax.ShapeDtypeStruct((NC, NPADD), jnp.float32),
    mesh=_sc_mesh,
    scratch_types=[
        pltpu.VMEM((NSTG, SSTG, K), jnp.int32),
        pltpu.VMEM((K,), jnp.float32),
        pltpu.VMEM((DEG_SLICE,), jnp.float32),
        pltpu.VMEM_SHARED((NPADD,), jnp.float32),
    ],
)
def _deg_kernel(eidx_hbm, out_hbm, dst_v, ones_v, zbuf, deg_sh):
    c = lax.axis_index("c")
    s = lax.axis_index("s")
    w = c * NS + s
    pltpu.sync_copy(eidx_hbm.at[1, w], dst_v)
    for i in range(K // 16):
        ones_v[pl.ds(i * 16, 16)] = jnp.ones((16,), jnp.float32)
    for i in range(DEG_SLICE // 16):
        zbuf[pl.ds(i * 16, 16)] = jnp.zeros((16,), jnp.float32)
    pltpu.sync_copy(zbuf, deg_sh.at[pl.ds(s * DEG_SLICE, DEG_SLICE)])
    plsc.subcore_barrier()

    def body(j, carry):
        pltpu.sync_copy(ones_v, deg_sh.at[dst_v.at[j // SSTG, j % SSTG]], add=True)
        return carry

    lax.fori_loop(0, NCHUNK, body, 0)
    plsc.subcore_barrier()
    pltpu.sync_copy(deg_sh.at[pl.ds(s * DEG_SLICE, DEG_SLICE)],
                    out_hbm.at[c, pl.ds(s * DEG_SLICE, DEG_SLICE)])


@functools.partial(
    pl.kernel,
    out_type=jax.ShapeDtypeStruct((NC, NPAD, DH), jnp.float32),
    mesh=_sc_mesh,
    scratch_types=[
        pltpu.VMEM((SSTG, K), jnp.int32),
        pltpu.VMEM((SSTG, K), jnp.int32),
        pltpu.VMEM((K, DH), jnp.float32),
        pltpu.VMEM((K, DH), jnp.float32),
        pltpu.VMEM_SHARED((NPAD, DH), jnp.float32),
        pltpu.SemaphoreType.DMA,
        pltpu.SemaphoreType.DMA,
        pltpu.SemaphoreType.DMA,
        pltpu.SemaphoreType.DMA,
    ],
)
def _agg_kernel(g_hbm, eidx_hbm, zeros_hbm, out_hbm,
                src_v, dst_v, r0, r1, acc_sh,
                sg0, sg1, ss0, ss1):
    c = lax.axis_index("c")
    s = lax.axis_index("s")
    w = c * NS + s
    pltpu.sync_copy(zeros_hbm,
                    acc_sh.at[pl.ds(s * ROWS_PER_TILE, ROWS_PER_TILE)])
    plsc.subcore_barrier()

    bufs = (r0, r1)
    sgs = (sg0, sg1)
    sss = (ss0, ss1)

    def start_g(j, b):
        pltpu.async_copy(g_hbm.at[src_v.at[j]], bufs[b], sgs[b])

    def wait_g(b):
        pltpu.make_async_copy(g_hbm.at[pl.ds(0, K)], bufs[b], sgs[b]).wait()

    def start_s(j, b):
        pltpu.async_copy(bufs[b], acc_sh.at[dst_v.at[j]], sss[b], add=True)

    def wait_s(b):
        pltpu.make_async_copy(bufs[b], acc_sh.at[dst_v.at[0]], sss[b]).wait()

    def stage(st, carry):
        pltpu.sync_copy(eidx_hbm.at[0, w, st], src_v)
        pltpu.sync_copy(eidx_hbm.at[1, w, st], dst_v)
        start_g(0, 0)
        start_g(1, 1)

        def pair(i, carry2):
            j = 2 * i
            wait_g(0); start_s(j, 0); wait_s(0); start_g(j + 2, 0)
            wait_g(1); start_s(j + 1, 1); wait_s(1); start_g(j + 3, 1)
            return carry2

        lax.fori_loop(0, (SSTG - 2) // 2, pair, 0)
        wait_g(0); start_s(SSTG - 2, 0); wait_s(0)
        wait_g(1); start_s(SSTG - 1, 1); wait_s(1)
        return carry

    lax.fori_loop(0, NSTG, stage, 0)
    plsc.subcore_barrier()
    pltpu.sync_copy(acc_sh.at[pl.ds(s * ROWS_PER_TILE, ROWS_PER_TILE)],
                    out_hbm.at[c, pl.ds(s * ROWS_PER_TILE, ROWS_PER_TILE)])


def _dinv(deg_ref):
    return lax.rsqrt(deg_ref[0] + deg_ref[1] + 1.0)


def _tc_pre_body(deg_ref, x_ref, w1_ref, g1_ref):
    g1_ref[...] = _dinv(deg_ref) * jnp.dot(
        x_ref[...], w1_ref[...], preferred_element_type=jnp.float32)


def _tc_mid_body(deg_ref, acc_ref, g1_ref, b1_ref, w2_ref, g2_ref):
    dinv = _dinv(deg_ref)
    a = acc_ref[0] + acc_ref[1] + g1_ref[...]
    h = jnp.maximum(dinv * a + b1_ref[...], 0.0)
    g2_ref[...] = dinv * jnp.dot(h, w2_ref[...],
                                 preferred_element_type=jnp.float32)


def _tc_post_body(deg_ref, acc_ref, g2_ref, b2_ref, out_ref):
    dinv = _dinv(deg_ref)
    a = acc_ref[0] + acc_ref[1] + g2_ref[...]
    out_ref[...] = jnp.maximum(dinv * a + b2_ref[...], 0.0)


_deg_spec = pl.BlockSpec((NC, R, 1), lambda i: (0, i, 0))
_row_spec = pl.BlockSpec((R, DH), lambda i: (i, 0))
_acc_spec = pl.BlockSpec((NC, R, DH), lambda i: (0, i, 0))
_w_spec = pl.BlockSpec((DH, DH), lambda i: (0, 0))
_b_spec = pl.BlockSpec((1, DH), lambda i: (0, 0))

_tc_pre = pl.pallas_call(
    _tc_pre_body,
    grid=(GRID,),
    in_specs=[_deg_spec, _row_spec, _w_spec],
    out_specs=_row_spec,
    out_shape=jax.ShapeDtypeStruct((N, DH), jnp.float32),
)

_tc_mid = pl.pallas_call(
    _tc_mid_body,
    grid=(GRID,),
    in_specs=[_deg_spec, _acc_spec, _row_spec, _b_spec, _w_spec],
    out_specs=_row_spec,
    out_shape=jax.ShapeDtypeStruct((N, DH), jnp.float32),
)

_tc_post = pl.pallas_call(
    _tc_post_body,
    grid=(GRID,),
    in_specs=[_deg_spec, _acc_spec, _row_spec, _b_spec],
    out_specs=_row_spec,
    out_shape=jax.ShapeDtypeStruct((N, DH), jnp.float32),
)


def kernel(x, edge_index, W1, b1, W2, b2):
    pad_cols = jnp.concatenate(
        [jnp.zeros((1, PAD_E), jnp.int32),
         jnp.full((1, PAD_E), N, jnp.int32)], axis=0)
    eidx = jnp.concatenate([edge_index, pad_cols],
                           axis=1).reshape(2, NW, NSTG, SSTG, K)
    b1r = b1.reshape(1, DH)
    b2r = b2.reshape(1, DH)
    zrows = jnp.zeros((ROWS_PER_TILE, DH), jnp.float32)

    degp = _deg_kernel(eidx).reshape(NC, NPADD, 1)
    g1 = _tc_pre(degp, x, W1)
    acc1 = _agg_kernel(g1, eidx, zrows)
    g2 = _tc_mid(degp, acc1, g1, b1r, W2)
    acc2 = _agg_kernel(g2, eidx, zrows)
    return _tc_post(degp, acc2, g2, b2r)


# speedup vs baseline: 3.8362x; 3.8362x over previous
import functools

import jax
import jax.numpy as jnp
from jax import lax
from jax.experimental import pallas as pl
from jax.experimental.pallas import tpu as pltpu
from jax.experimental.pallas import tpu_sc as plsc

N = 10000
DH = 128
E = 320000
NC = 2
NS = 16
NW = NC * NS
EPW = E // NW
K = 80
NCHUNK = EPW // K
SSTG = 25
NSTG = NCHUNK // SSTG
NPAIR = (SSTG - 1) // 2
NPAD = 10240
DEG_SLICE = NPAD // NS
ROWS_PER_TILE = NPAD // NS
RZ = 16

R = 2000
GRID = N // R

_sc_mesh = plsc.VectorSubcoreMesh(core_axis_name="c", subcore_axis_name="s")


@functools.partial(
    pl.kernel,
    out_type=jax.ShapeDtypeStruct((NC, NPAD), jnp.float32),
    mesh=_sc_mesh,
    scratch_types=[
        pltpu.VMEM((NSTG, SSTG, K), jnp.int32),
        pltpu.VMEM((K,), jnp.float32),
        pltpu.VMEM((DEG_SLICE,), jnp.float32),
        pltpu.VMEM_SHARED((NPAD,), jnp.float32),
    ],
)
def _deg_kernel(eidx_hbm, out_hbm, dst_v, ones_v, zbuf, deg_sh):
    c = lax.axis_index("c")
    s = lax.axis_index("s")
    w = c * NS + s
    pltpu.sync_copy(eidx_hbm.at[1, w], dst_v)
    for i in range(K // 16):
        ones_v[pl.ds(i * 16, 16)] = jnp.ones((16,), jnp.float32)
    for i in range(DEG_SLICE // 16):
        zbuf[pl.ds(i * 16, 16)] = jnp.zeros((16,), jnp.float32)
    pltpu.sync_copy(zbuf, deg_sh.at[pl.ds(s * DEG_SLICE, DEG_SLICE)])
    plsc.subcore_barrier()

    def body(j, carry):
        pltpu.sync_copy(ones_v, deg_sh.at[dst_v.at[j // SSTG, j % SSTG]], add=True)
        return carry

    lax.fori_loop(0, NCHUNK, body, 0)
    plsc.subcore_barrier()
    pltpu.sync_copy(deg_sh.at[pl.ds(s * DEG_SLICE, DEG_SLICE)],
                    out_hbm.at[c, pl.ds(s * DEG_SLICE, DEG_SLICE)])


@functools.partial(
    pl.kernel,
    out_type=jax.ShapeDtypeStruct((NC, NPAD, DH), jnp.float32),
    mesh=_sc_mesh,
    scratch_types=[
        pltpu.VMEM((SSTG, K), jnp.int32),
        pltpu.VMEM((SSTG, K), jnp.int32),
        pltpu.VMEM((K, DH), jnp.float32),
        pltpu.VMEM((K, DH), jnp.float32),
        pltpu.VMEM((K, DH), jnp.float32),
        pltpu.VMEM_SHARED((NPAD, DH), jnp.float32),
        pltpu.SemaphoreType.DMA,
        pltpu.SemaphoreType.DMA,
        pltpu.SemaphoreType.DMA,
        pltpu.SemaphoreType.DMA,
        pltpu.SemaphoreType.DMA,
        pltpu.SemaphoreType.DMA,
    ],
)
def _agg_kernel(g_hbm, eidx_hbm, zeros_hbm, out_hbm,
                src_v, dst_v, r0, r1, r2, acc_sh,
                sg0, sg1, sg2, ss0, ss1, ss2):
    c = lax.axis_index("c")
    s = lax.axis_index("s")
    w = c * NS + s
    pltpu.sync_copy(zeros_hbm,
                    acc_sh.at[pl.ds(s * ROWS_PER_TILE, ROWS_PER_TILE)])
    plsc.subcore_barrier()

    bufs = (r0, r1, r2)
    sgs = (sg0, sg1, sg2)
    sss = (ss0, ss1, ss2)

    def start_g(j, b):
        pltpu.async_copy(g_hbm.at[src_v.at[j]], bufs[b], sgs[b])

    def wait_g(b):
        pltpu.make_async_copy(g_hbm.at[pl.ds(0, K)], bufs[b], sgs[b]).wait()

    def start_s(j, b):
        pltpu.async_copy(bufs[b], acc_sh.at[dst_v.at[j]], sss[b], add=True)

    def wait_s(b):
        pltpu.make_async_copy(bufs[b], acc_sh.at[dst_v.at[0]], sss[b]).wait()

    def stage(st, carry):
        pltpu.sync_copy(eidx_hbm.at[0, w, st], src_v)
        pltpu.sync_copy(eidx_hbm.at[1, w, st], dst_v)
        for b in range(3):
            start_g(b, b)

        def group(i, carry2):
            j0 = 3 * i
            for b in range(3):
                wait_g(b)
                start_s(j0 + b, b)
                wait_s(b)
                start_g(j0 + b + 3, b)
            return carry2

        lax.fori_loop(0, (SSTG - 4) // 3, group, 0)
        wait_g(0); start_s(SSTG - 4, 0); wait_s(0); start_g(SSTG - 1, 0)
        wait_g(1); start_s(SSTG - 3, 1); wait_s(1)
        wait_g(2); start_s(SSTG - 2, 2); wait_s(2)
        wait_g(0); start_s(SSTG - 1, 0); wait_s(0)
        return carry

    lax.fori_loop(0, NSTG, stage, 0)
    plsc.subcore_barrier()
    pltpu.sync_copy(acc_sh.at[pl.ds(s * ROWS_PER_TILE, ROWS_PER_TILE)],
                    out_hbm.at[c, pl.ds(s * ROWS_PER_TILE, ROWS_PER_TILE)])


def _dinv(deg_ref):
    return lax.rsqrt(deg_ref[0] + deg_ref[1] + 1.0)


def _tc_pre_body(deg_ref, x_ref, w1_ref, g1_ref):
    g1_ref[...] = _dinv(deg_ref) * jnp.dot(
        x_ref[...], w1_ref[...], preferred_element_type=jnp.float32)


def _tc_mid_body(deg_ref, acc_ref, g1_ref, b1_ref, w2_ref, g2_ref):
    dinv = _dinv(deg_ref)
    a = acc_ref[0] + acc_ref[1] + g1_ref[...]
    h = jnp.maximum(dinv * a + b1_ref[...], 0.0)
    g2_ref[...] = dinv * jnp.dot(h, w2_ref[...],
                                 preferred_element_type=jnp.float32)


def _tc_post_body(deg_ref, acc_ref, g2_ref, b2_ref, out_ref):
    dinv = _dinv(deg_ref)
    a = acc_ref[0] + acc_ref[1] + g2_ref[...]
    out_ref[...] = jnp.maximum(dinv * a + b2_ref[...], 0.0)


_deg_spec = pl.BlockSpec((NC, R, 1), lambda i: (0, i, 0))
_row_spec = pl.BlockSpec((R, DH), lambda i: (i, 0))
_acc_spec = pl.BlockSpec((NC, R, DH), lambda i: (0, i, 0))
_w_spec = pl.BlockSpec((DH, DH), lambda i: (0, 0))
_b_spec = pl.BlockSpec((1, DH), lambda i: (0, 0))

_tc_pre = pl.pallas_call(
    _tc_pre_body,
    grid=(GRID,),
    in_specs=[_deg_spec, _row_spec, _w_spec],
    out_specs=_row_spec,
    out_shape=jax.ShapeDtypeStruct((N, DH), jnp.float32),
)

_tc_mid = pl.pallas_call(
    _tc_mid_body,
    grid=(GRID,),
    in_specs=[_deg_spec, _acc_spec, _row_spec, _b_spec, _w_spec],
    out_specs=_row_spec,
    out_shape=jax.ShapeDtypeStruct((N, DH), jnp.float32),
)

_tc_post = pl.pallas_call(
    _tc_post_body,
    grid=(GRID,),
    in_specs=[_deg_spec, _acc_spec, _row_spec, _b_spec],
    out_specs=_row_spec,
    out_shape=jax.ShapeDtypeStruct((N, DH), jnp.float32),
)


def kernel(x, edge_index, W1, b1, W2, b2):
    eidx = edge_index.reshape(2, NW, NSTG, SSTG, K)
    b1r = b1.reshape(1, DH)
    b2r = b2.reshape(1, DH)
    zrows = jnp.zeros((ROWS_PER_TILE, DH), jnp.float32)

    degp = _deg_kernel(eidx).reshape(NC, NPAD, 1)
    g1 = _tc_pre(degp, x, W1)
    acc1 = _agg_kernel(g1, eidx, zrows)
    g2 = _tc_mid(degp, acc1, g1, b1r, W2)
    acc2 = _agg_kernel(g2, eidx, zrows)
    return _tc_post(degp, acc2, g2, b2r)
